# baseline (speedup 1.0000x reference)
import functools

import jax
import jax.numpy as jnp
from jax import lax
from jax.experimental import pallas as pl
from jax.experimental.pallas import tpu as pltpu
from jax.experimental.pallas import tpu_sc as plsc

NUM_GROUPS = 16
B = 16384
C = 128
EPS = 1e-05

NC = 2
NS = 16
NW = NC * NS
L = 16

ROWS = B // NW
NV = C // L
NCHUNK = 4
CROWS = ROWS // NCHUNK
GROUP_ELEMS = float(B * (C // NUM_GROUPS))

_MESH = plsc.VectorSubcoreMesh(core_axis_name="c", subcore_axis_name="s")


def _newton_rsqrt(v):
    i = lax.bitcast_convert_type(v, jnp.int32)
    y = lax.bitcast_convert_type(jnp.int32(0x5F3759DF) - (i >> 1), jnp.float32)
    half = v * 0.5
    for _ in range(4):
        y = y * (1.5 - half * y * y)
    return y


def _extract(v, k):
    return lax.squeeze(lax.slice(v, (k,), (k + 1,)), (0,))


@functools.partial(
    pl.kernel,
    out_type=jax.ShapeDtypeStruct((B, C), jnp.float32),
    mesh=_MESH,
    scratch_types=[
        [pltpu.VMEM((CROWS, C), jnp.float32) for _ in range(NCHUNK)],
        [pltpu.VMEM((CROWS, C), jnp.float32) for _ in range(2)],
        pltpu.VMEM((2, C), jnp.float32),
        pltpu.VMEM((NS, 2, C), jnp.float32),
        pltpu.VMEM_SHARED((NS, 2, C), jnp.float32),
        [pltpu.SemaphoreType.DMA for _ in range(NCHUNK)],
        [pltpu.SemaphoreType.DMA for _ in range(2)],
        [pltpu.SemaphoreType.DMA for _ in range(NCHUNK)],
    ],
)
def _gbn(x_hbm, out_hbm, own, peer, part_v, all_v, shared,
         own_sems, peer_sems, out_sems):
    cid = lax.axis_index("c")
    sid = lax.axis_index("s")
    wid = sid * NC + cid
    wid_peer = sid * NC + (1 - cid)
    r_own = wid * ROWS
    r_peer = wid_peer * ROWS

    own_h = [
        pltpu.async_copy(
            x_hbm.at[pl.ds(r_own + c * CROWS, CROWS), :], own[c], own_sems[c]
        )
        for c in range(NCHUNK)
    ]
    peer_h = [
        pltpu.async_copy(
            x_hbm.at[pl.ds(r_peer + c * CROWS, CROWS), :],
            peer[c % 2],
            peer_sems[c % 2],
        )
        for c in range(2)
    ]

    zero = jnp.zeros((L,), jnp.float32)

    def make_acc(buf):
        def acc_body(i, carry):
            sums = list(carry[:NV])
            sqs = list(carry[NV:])
            for j in range(NV):
                v = buf[i, pl.ds(j * L, L)]
                sums[j] = sums[j] + v
                sqs[j] = sqs[j] + v * v
            return tuple(sums) + tuple(sqs)

        return acc_body

    carry = (zero,) * (2 * NV)
    for c in range(NCHUNK):
        own_h[c].wait()
        carry = lax.fori_loop(0, CROWS, make_acc(own[c]), carry, unroll=2)
    for c in range(NCHUNK):
        peer_h[c].wait()
        buf = peer[c % 2]
        carry = lax.fori_loop(0, CROWS, make_acc(buf), carry, unroll=2)
        nxt = c + 2
        if nxt < NCHUNK:
            peer_h.append(
                pltpu.async_copy(
                    x_hbm.at[pl.ds(r_peer + nxt * CROWS, CROWS), :],
                    peer[nxt % 2],
                    peer_sems[nxt % 2],
                )
            )

    for j in range(NV):
        part_v[0, pl.ds(j * L, L)] = carry[j]
        part_v[1, pl.ds(j * L, L)] = carry[NV + j]

    pltpu.sync_copy(part_v, shared.at[sid])
    plsc.subcore_barrier()
    pltpu.sync_copy(shared, all_v)

    tot = [zero] * (2 * NV)
    for t in range(NS):
        for j in range(NV):
            tot[j] = tot[j] + all_v[t, 0, pl.ds(j * L, L)]
            tot[NV + j] = tot[NV + j] + all_v[t, 1, pl.ds(j * L, L)]

    lane = lax.iota(jnp.int32, L)
    lo_mask = lane < 8
    zvec = jnp.zeros((L,), jnp.float32)
    inv_n = 1.0 / GROUP_ELEMS

    def half_sums(v):
        el = [_extract(v, k) for k in range(L)]

        def tree(vals):
            while len(vals) > 1:
                vals = [a + b for a, b in zip(vals[::2], vals[1::2])]
            return vals[0]

        return tree(el[:8]), tree(el[8:])

    mean_vecs = []
    rstd_vecs = []
    for j in range(NV):
        s_lo, s_hi = half_sums(tot[j])
        q_lo, q_hi = half_sums(tot[NV + j])
        m_lo = s_lo * inv_n
        m_hi = s_hi * inv_n
        r_lo = _newton_rsqrt(q_lo * inv_n - m_lo * m_lo + EPS)
        r_hi = _newton_rsqrt(q_hi * inv_n - m_hi * m_hi + EPS)
        mean_vecs.append(jnp.where(lo_mask, zvec + m_lo, zvec + m_hi))
        rstd_vecs.append(jnp.where(lo_mask, zvec + r_lo, zvec + r_hi))

    SLAB = 32
    out_h = []
    si = 0
    for c in range(NCHUNK):
        buf = own[c]

        def norm_body(i, carry, buf=buf):
            for j in range(NV):
                sl = pl.ds(j * L, L)
                buf[i, sl] = (buf[i, sl] - mean_vecs[j]) * rstd_vecs[j]
            return carry

        for s0 in range(0, CROWS, SLAB):
            lax.fori_loop(s0, s0 + SLAB, norm_body, 0, unroll=2)
            out_h.append(
                pltpu.async_copy(
                    buf.at[pl.ds(s0, SLAB), :],
                    out_hbm.at[pl.ds(r_own + c * CROWS + s0, SLAB), :],
                    out_sems[si % NCHUNK],
                )
            )
            si += 1

    for h in out_h:
        h.wait()


def kernel(x, channel_groups):
    del channel_groups
    return _gbn(x)[:, :, None]


# speedup vs baseline: 1.0286x; 1.0286x over previous
import functools

import jax
import jax.numpy as jnp
from jax import lax
from jax.experimental import pallas as pl
from jax.experimental.pallas import tpu as pltpu
from jax.experimental.pallas import tpu_sc as plsc

NUM_GROUPS = 16
B = 16384
C = 128
EPS = 1e-05

NC = 2
NS = 16
NW = NC * NS
L = 16

ROWS = B // NW
NV = C // L
NCHUNK = 4
CROWS = ROWS // NCHUNK
GROUP_ELEMS = float(B * (C // NUM_GROUPS))

_MESH = plsc.VectorSubcoreMesh(core_axis_name="c", subcore_axis_name="s")


def _newton_rsqrt(v):
    i = lax.bitcast_convert_type(v, jnp.int32)
    y = lax.bitcast_convert_type(jnp.int32(0x5F3759DF) - (i >> 1), jnp.float32)
    half = v * 0.5
    for _ in range(4):
        y = y * (1.5 - half * y * y)
    return y


def _extract(v, k):
    return lax.squeeze(lax.slice(v, (k,), (k + 1,)), (0,))


@functools.partial(
    pl.kernel,
    out_type=jax.ShapeDtypeStruct((B, C), jnp.float32),
    mesh=_MESH,
    scratch_types=[
        [pltpu.VMEM((CROWS, C), jnp.float32) for _ in range(NCHUNK)],
        [pltpu.VMEM((CROWS, C), jnp.float32) for _ in range(2)],
        pltpu.VMEM((2, C), jnp.float32),
        pltpu.VMEM((NS, 2, C), jnp.float32),
        pltpu.VMEM_SHARED((NS, 2, C), jnp.float32),
        [pltpu.SemaphoreType.DMA for _ in range(NCHUNK)],
        [pltpu.SemaphoreType.DMA for _ in range(2)],
        [pltpu.SemaphoreType.DMA for _ in range(NCHUNK)],
    ],
)
def _gbn(x_hbm, out_hbm, own, peer, part_v, all_v, shared,
         own_sems, peer_sems, out_sems):
    cid = lax.axis_index("c")
    sid = lax.axis_index("s")
    wid = sid * NC + cid
    wid_peer = sid * NC + (1 - cid)
    r_own = wid * ROWS
    r_peer = wid_peer * ROWS

    own_h = [
        pltpu.async_copy(
            x_hbm.at[pl.ds(r_own + c * CROWS, CROWS), :], own[c], own_sems[c]
        )
        for c in range(NCHUNK)
    ]
    peer_h = [
        pltpu.async_copy(
            x_hbm.at[pl.ds(r_peer + c * CROWS, CROWS), :],
            peer[c % 2],
            peer_sems[c % 2],
        )
        for c in range(2)
    ]

    zero = jnp.zeros((L,), jnp.float32)

    def make_acc(buf):
        def acc_body(i, carry):
            sums = list(carry[:NV])
            sqs = list(carry[NV:])
            for j in range(NV):
                v = buf[i, pl.ds(j * L, L)]
                sums[j] = sums[j] + v
                sqs[j] = sqs[j] + v * v
            return tuple(sums) + tuple(sqs)

        return acc_body

    carry = (zero,) * (2 * NV)
    for c in range(NCHUNK):
        own_h[c].wait()
        carry = lax.fori_loop(0, CROWS, make_acc(own[c]), carry, unroll=2)
    for c in range(NCHUNK):
        peer_h[c].wait()
        buf = peer[c % 2]
        carry = lax.fori_loop(0, CROWS, make_acc(buf), carry, unroll=2)
        nxt = c + 2
        if nxt < NCHUNK:
            peer_h.append(
                pltpu.async_copy(
                    x_hbm.at[pl.ds(r_peer + nxt * CROWS, CROWS), :],
                    peer[nxt % 2],
                    peer_sems[nxt % 2],
                )
            )

    for j in range(NV):
        part_v[0, pl.ds(j * L, L)] = carry[j]
        part_v[1, pl.ds(j * L, L)] = carry[NV + j]

    pltpu.sync_copy(part_v, shared.at[sid])
    plsc.subcore_barrier()
    pltpu.sync_copy(shared, all_v)

    tot = [zero] * (2 * NV)
    for t in range(NS):
        for j in range(NV):
            tot[j] = tot[j] + all_v[t, 0, pl.ds(j * L, L)]
            tot[NV + j] = tot[NV + j] + all_v[t, 1, pl.ds(j * L, L)]

    lane = lax.iota(jnp.int32, L)
    lo_mask = lane < 8
    zvec = jnp.zeros((L,), jnp.float32)
    inv_n = 1.0 / GROUP_ELEMS

    def half_sums(v):
        el = [_extract(v, k) for k in range(L)]

        def tree(vals):
            while len(vals) > 1:
                vals = [a + b for a, b in zip(vals[::2], vals[1::2])]
            return vals[0]

        return tree(el[:8]), tree(el[8:])

    mean_vecs = []
    rstd_vecs = []
    for j in range(NV):
        s_lo, s_hi = half_sums(tot[j])
        q_lo, q_hi = half_sums(tot[NV + j])
        m_lo = s_lo * inv_n
        m_hi = s_hi * inv_n
        r_lo = _newton_rsqrt(q_lo * inv_n - m_lo * m_lo + EPS)
        r_hi = _newton_rsqrt(q_hi * inv_n - m_hi * m_hi + EPS)
        mean_vecs.append(jnp.where(lo_mask, zvec + m_lo, zvec + m_hi))
        rstd_vecs.append(jnp.where(lo_mask, zvec + r_lo, zvec + r_hi))

    out_h = []
    for c in range(NCHUNK):
        buf = own[c]

        def norm_body(i, carry, buf=buf):
            for j in range(NV):
                sl = pl.ds(j * L, L)
                buf[i, sl] = (buf[i, sl] - mean_vecs[j]) * rstd_vecs[j]
            return carry

        lax.fori_loop(0, CROWS, norm_body, 0, unroll=2)
        out_h.append(
            pltpu.async_copy(
                buf, out_hbm.at[pl.ds(r_own + c * CROWS, CROWS), :], out_sems[c]
            )
        )

    for h in out_h:
        h.wait()


def kernel(x, channel_groups):
    del channel_groups
    return _gbn(x)[:, :, None]
